# Initial kernel scaffold; baseline (speedup 1.0000x reference)
import jax
import jax.numpy as jnp
from jax.experimental import pallas as pl


def kernel(user_id, movie_title, user_table, movie_table, W1, b1, W2, b2, W3, b3):
    raise NotImplementedError("write your pallas kernel here")


# speedup vs baseline: 2.7693x; 2.7693x over previous
import jax
import jax.numpy as jnp
from jax import lax
from jax.experimental import pallas as pl
from jax.experimental.pallas import tpu as pltpu
from jax.experimental.pallas import tpu_sc as plsc

B = 16384
V = 100001
D = 32
H1 = 256
H2 = 64

NC = 2
NS = 16
NW = NC * NS
OCH = 4096
NOCH = B // OCH

_LANES = 16


def _sc_gather_body(tabT_hbm, idx_hbm, out_hbm, idx_v, row_v, ob0, ob1,
                    sem_row, sem_idx, sem_o0, sem_o1):
    d = lax.axis_index("s") * NC + lax.axis_index("c")
    row_cp = pltpu.async_copy(tabT_hbm.at[d], row_v, sem_row)
    idx_cp = pltpu.async_copy(idx_hbm, idx_v, sem_idx)
    row_cp.wait()
    idx_cp.wait()

    obufs = (ob0, ob1)
    osems = (sem_o0, sem_o1)
    pending = [None, None]
    for c in range(NOCH):
        ob = obufs[c % 2]
        if pending[c % 2] is not None:
            pending[c % 2].wait()

        def _inner(i, _, _c=c, _ob=ob):
            off = _c * OCH + i * _LANES
            vals = plsc.load_gather(row_v, [idx_v[pl.ds(off, _LANES)]])
            _ob[pl.ds(i * _LANES, _LANES)] = vals
            return 0

        lax.fori_loop(0, OCH // _LANES, _inner, 0, unroll=8)
        pending[c % 2] = pltpu.async_copy(
            ob, out_hbm.at[d, pl.ds(c * OCH, OCH)], osems[c % 2])
    for cp in pending:
        cp.wait()


def _sc_gather(tabT, idx):
    return pl.kernel(
        _sc_gather_body,
        out_type=jax.ShapeDtypeStruct((D, B), jnp.float32),
        mesh=plsc.VectorSubcoreMesh(core_axis_name="c", subcore_axis_name="s"),
        compiler_params=pltpu.CompilerParams(needs_layout_passes=False),
        scratch_types=[
            pltpu.VMEM((B,), jnp.int32),
            pltpu.VMEM((V,), jnp.float32),
            pltpu.VMEM((OCH,), jnp.float32),
            pltpu.VMEM((OCH,), jnp.float32),
            pltpu.SemaphoreType.DMA,
            pltpu.SemaphoreType.DMA,
            pltpu.SemaphoreType.DMA,
            pltpu.SemaphoreType.DMA,
        ],
    )(tabT, idx)


def _mlp_body(xuT_ref, xmT_ref, w1tu_ref, w1tm_ref, b1_ref, w2t_ref, b2_ref,
              w3_ref, b3_ref, outT_ref):
    h1 = lax.dot_general(w1tu_ref[...], xuT_ref[...],
                         (((1,), (0,)), ((), ())),
                         preferred_element_type=jnp.float32)
    h1 += lax.dot_general(w1tm_ref[...], xmT_ref[...],
                          (((1,), (0,)), ((), ())),
                          preferred_element_type=jnp.float32)
    h1 = jnp.maximum(h1 + b1_ref[...], 0.0)
    h2 = lax.dot_general(w2t_ref[...], h1, (((1,), (0,)), ((), ())),
                         preferred_element_type=jnp.float32)
    h2 = jnp.maximum(h2 + b2_ref[...], 0.0)
    outT_ref[...] = (jnp.sum(h2 * w3_ref[...], axis=0, keepdims=True)
                     + b3_ref[0])


BLK = 2048


def _mlp(xuT, xmT, W1, b1, W2, b2, W3, b3):
    W1T = W1.T
    grid = B // BLK
    outT = pl.pallas_call(
        _mlp_body,
        grid=(grid,),
        in_specs=[
            pl.BlockSpec((D, BLK), lambda i: (0, i)),
            pl.BlockSpec((D, BLK), lambda i: (0, i)),
            pl.BlockSpec((H1, D), lambda i: (0, 0)),
            pl.BlockSpec((H1, D), lambda i: (0, 0)),
            pl.BlockSpec((H1, 1), lambda i: (0, 0)),
            pl.BlockSpec((H2, H1), lambda i: (0, 0)),
            pl.BlockSpec((H2, 1), lambda i: (0, 0)),
            pl.BlockSpec((H2, 1), lambda i: (0, 0)),
            pl.BlockSpec(memory_space=pltpu.SMEM),
        ],
        out_specs=pl.BlockSpec((1, BLK), lambda i: (0, i)),
        out_shape=jax.ShapeDtypeStruct((1, B), jnp.float32),
    )(xuT, xmT, W1T[:, :D], W1T[:, D:], b1.reshape(H1, 1), W2.T,
      b2.reshape(H2, 1), W3, b3)
    return outT.reshape(B, 1)


def kernel(user_id, movie_title, user_table, movie_table,
           W1, b1, W2, b2, W3, b3):
    uid = user_id.astype(jnp.int32)
    mid = movie_title.astype(jnp.int32)
    xuT = _sc_gather(user_table.T, uid)
    xmT = _sc_gather(movie_table.T, mid)
    return _mlp(xuT, xmT, W1, b1, W2, b2, W3, b3)


# speedup vs baseline: 3.1484x; 1.1369x over previous
import jax
import jax.numpy as jnp
from jax import lax
from jax.experimental import pallas as pl
from jax.experimental.pallas import tpu as pltpu
from jax.experimental.pallas import tpu_sc as plsc

B = 16384
V = 100001
D = 32
H1 = 256
H2 = 64

NC = 2
NS = 16
NW = NC * NS
OCH = 4096
NOCH = B // OCH

_LANES = 16


def _sc_gather_body(utabT_hbm, mtabT_hbm, uid_hbm, mid_hbm, out_hbm,
                    idx_v, row_v, ob0, ob1,
                    sem_row, sem_idx, sem_o0, sem_o1):
    d = lax.axis_index("s") * NC + lax.axis_index("c")
    obufs = (ob0, ob1)
    osems = (sem_o0, sem_o1)
    pending = [None, None]

    for tab_hbm, ids_hbm, row_base in ((utabT_hbm, uid_hbm, 0),
                                       (mtabT_hbm, mid_hbm, D)):
        row_cp = pltpu.async_copy(tab_hbm.at[d], row_v, sem_row)
        idx_cp = pltpu.async_copy(ids_hbm, idx_v, sem_idx)
        row_cp.wait()
        idx_cp.wait()
        for c in range(NOCH):
            ob = obufs[c % 2]
            if pending[c % 2] is not None:
                pending[c % 2].wait()

            def _inner(i, _, _c=c, _ob=ob):
                off = _c * OCH + i * _LANES
                vals = plsc.load_gather(row_v, [idx_v[pl.ds(off, _LANES)]])
                _ob[pl.ds(i * _LANES, _LANES)] = vals
                return 0

            lax.fori_loop(0, OCH // _LANES, _inner, 0, unroll=8)
            pending[c % 2] = pltpu.async_copy(
                ob, out_hbm.at[row_base + d, pl.ds(c * OCH, OCH)],
                osems[c % 2])
    for cp in pending:
        cp.wait()


def _sc_gather(utabT, mtabT, uid, mid):
    return pl.kernel(
        _sc_gather_body,
        out_type=jax.ShapeDtypeStruct((2 * D, B), jnp.float32),
        mesh=plsc.VectorSubcoreMesh(core_axis_name="c", subcore_axis_name="s"),
        compiler_params=pltpu.CompilerParams(needs_layout_passes=False),
        scratch_types=[
            pltpu.VMEM((B,), jnp.int32),
            pltpu.VMEM((V,), jnp.float32),
            pltpu.VMEM((OCH,), jnp.float32),
            pltpu.VMEM((OCH,), jnp.float32),
            pltpu.SemaphoreType.DMA,
            pltpu.SemaphoreType.DMA,
            pltpu.SemaphoreType.DMA,
            pltpu.SemaphoreType.DMA,
        ],
    )(utabT, mtabT, uid, mid)


def _mlp_body(xT_ref, w1t_ref, b1_ref, w2t_ref, b2_ref,
              w3_ref, b3_ref, outT_ref):
    x = xT_ref[...].astype(jnp.bfloat16)
    w1 = w1t_ref[...].astype(jnp.bfloat16)
    h1 = lax.dot_general(w1, x, (((1,), (0,)), ((), ())),
                         preferred_element_type=jnp.float32)
    h1 = jnp.maximum(h1 + b1_ref[...], 0.0).astype(jnp.bfloat16)
    w2 = w2t_ref[...].astype(jnp.bfloat16)
    h2 = lax.dot_general(w2, h1, (((1,), (0,)), ((), ())),
                         preferred_element_type=jnp.float32)
    h2 = jnp.maximum(h2 + b2_ref[...], 0.0)
    outT_ref[...] = (jnp.sum(h2 * w3_ref[...], axis=0, keepdims=True)
                     + b3_ref[0])


BLK = 4096


def _mlp(xT, W1, b1, W2, b2, W3, b3):
    grid = B // BLK
    outT = pl.pallas_call(
        _mlp_body,
        grid=(grid,),
        in_specs=[
            pl.BlockSpec((2 * D, BLK), lambda i: (0, i)),
            pl.BlockSpec((H1, 2 * D), lambda i: (0, 0)),
            pl.BlockSpec((H1, 1), lambda i: (0, 0)),
            pl.BlockSpec((H2, H1), lambda i: (0, 0)),
            pl.BlockSpec((H2, 1), lambda i: (0, 0)),
            pl.BlockSpec((H2, 1), lambda i: (0, 0)),
            pl.BlockSpec(memory_space=pltpu.SMEM),
        ],
        out_specs=pl.BlockSpec((1, BLK), lambda i: (0, i)),
        out_shape=jax.ShapeDtypeStruct((1, B), jnp.float32),
    )(xT, W1.T, b1.reshape(H1, 1), W2.T, b2.reshape(H2, 1), W3, b3)
    return outT.reshape(B, 1)


def kernel(user_id, movie_title, user_table, movie_table,
           W1, b1, W2, b2, W3, b3):
    uid = user_id.astype(jnp.int32)
    mid = movie_title.astype(jnp.int32)
    xT = _sc_gather(user_table.T, movie_table.T, uid, mid)
    return _mlp(xT, W1, b1, W2, b2, W3, b3)


# speedup vs baseline: 4.1553x; 1.3198x over previous
import jax
import jax.numpy as jnp
from jax import lax
from jax.experimental import pallas as pl
from jax.experimental.pallas import tpu as pltpu
from jax.experimental.pallas import tpu_sc as plsc

B = 16384
V = 100001
D = 32
H1 = 256
H2 = 64

NC = 2
NS = 16
NW = NC * NS
OCH = 4096
NOCH = B // OCH

_LANES = 16


def _sc_gather_body(utabT_hbm, mtabT_hbm, uid_hbm, mid_hbm, out_hbm,
                    idx_v, row_v, ob0, ob1,
                    sem_row, sem_idx, sem_o0, sem_o1):
    d = lax.axis_index("s") * NC + lax.axis_index("c")
    obufs = (ob0, ob1)
    osems = (sem_o0, sem_o1)
    pending = [None, None]

    for tab_hbm, ids_hbm, row_base in ((utabT_hbm, uid_hbm, 0),
                                       (mtabT_hbm, mid_hbm, D)):
        row_cp = pltpu.async_copy(tab_hbm.at[d], row_v, sem_row)
        idx_cp = pltpu.async_copy(ids_hbm, idx_v, sem_idx)
        row_cp.wait()
        idx_cp.wait()
        for c in range(NOCH):
            ob = obufs[c % 2]
            if pending[c % 2] is not None:
                pending[c % 2].wait()

            @plsc.parallel_loop(c * OCH, (c + 1) * OCH, step=_LANES,
                                unroll=8)
            def _inner(off, _c=c, _ob=ob):
                vals = plsc.load_gather(row_v, [idx_v[pl.ds(off, _LANES)]])
                _ob[pl.ds(off - _c * OCH, _LANES)] = vals
            pending[c % 2] = pltpu.async_copy(
                ob, out_hbm.at[row_base + d, pl.ds(c * OCH, OCH)],
                osems[c % 2])
    for cp in pending:
        cp.wait()


def _sc_gather(utabT, mtabT, uid, mid):
    return pl.kernel(
        _sc_gather_body,
        out_type=jax.ShapeDtypeStruct((2 * D, B), jnp.float32),
        mesh=plsc.VectorSubcoreMesh(core_axis_name="c", subcore_axis_name="s"),
        compiler_params=pltpu.CompilerParams(needs_layout_passes=False),
        scratch_types=[
            pltpu.VMEM((B,), jnp.int32),
            pltpu.VMEM((V,), jnp.float32),
            pltpu.VMEM((OCH,), jnp.float32),
            pltpu.VMEM((OCH,), jnp.float32),
            pltpu.SemaphoreType.DMA,
            pltpu.SemaphoreType.DMA,
            pltpu.SemaphoreType.DMA,
            pltpu.SemaphoreType.DMA,
        ],
    )(utabT, mtabT, uid, mid)


def _mlp_body(xT_ref, w1t_ref, b1_ref, w2t_ref, b2_ref,
              w3_ref, b3_ref, outT_ref):
    x = xT_ref[...].astype(jnp.bfloat16)
    w1 = w1t_ref[...].astype(jnp.bfloat16)
    h1 = lax.dot_general(w1, x, (((1,), (0,)), ((), ())),
                         preferred_element_type=jnp.float32)
    h1 = jnp.maximum(h1 + b1_ref[...], 0.0).astype(jnp.bfloat16)
    w2 = w2t_ref[...].astype(jnp.bfloat16)
    h2 = lax.dot_general(w2, h1, (((1,), (0,)), ((), ())),
                         preferred_element_type=jnp.float32)
    h2 = jnp.maximum(h2 + b2_ref[...], 0.0)
    outT_ref[...] = (jnp.sum(h2 * w3_ref[...], axis=0, keepdims=True)
                     + b3_ref[0])


BLK = 4096


def _mlp(xT, W1, b1, W2, b2, W3, b3):
    grid = B // BLK
    outT = pl.pallas_call(
        _mlp_body,
        grid=(grid,),
        in_specs=[
            pl.BlockSpec((2 * D, BLK), lambda i: (0, i)),
            pl.BlockSpec((H1, 2 * D), lambda i: (0, 0)),
            pl.BlockSpec((H1, 1), lambda i: (0, 0)),
            pl.BlockSpec((H2, H1), lambda i: (0, 0)),
            pl.BlockSpec((H2, 1), lambda i: (0, 0)),
            pl.BlockSpec((H2, 1), lambda i: (0, 0)),
            pl.BlockSpec(memory_space=pltpu.SMEM),
        ],
        out_specs=pl.BlockSpec((1, BLK), lambda i: (0, i)),
        out_shape=jax.ShapeDtypeStruct((1, B), jnp.float32),
    )(xT, W1.T, b1.reshape(H1, 1), W2.T, b2.reshape(H2, 1), W3, b3)
    return outT.reshape(B, 1)


def kernel(user_id, movie_title, user_table, movie_table,
           W1, b1, W2, b2, W3, b3):
    uid = user_id.astype(jnp.int32)
    mid = movie_title.astype(jnp.int32)
    xT = _sc_gather(user_table.T, movie_table.T, uid, mid)
    return _mlp(xT, W1, b1, W2, b2, W3, b3)
